# Initial kernel scaffold; baseline (speedup 1.0000x reference)
import jax
import jax.numpy as jnp
from jax.experimental import pallas as pl


def kernel(x):
    raise NotImplementedError("write your pallas kernel here")


# speedup vs baseline: 6.3351x; 6.3351x over previous
import numpy as np
import jax
import jax.numpy as jnp
from jax.experimental import pallas as pl

_PS = 16
_H = 512
_W = 512
_HP = _H // _PS
_WP = _W // _PS
_TOTAL = _HP * _WP
_NUM_MASK = int(0.75 * _TOTAL)
_MASK_VALUE = 0.5


def _full_mask() -> np.ndarray:
    perm = np.asarray(jax.random.permutation(jax.random.key(42), _TOTAL))
    patch_mask = np.zeros(_TOTAL, dtype=bool)
    patch_mask[perm[:_NUM_MASK]] = True
    grid2d = patch_mask.reshape(_HP, _WP)
    return np.repeat(np.repeat(grid2d, _PS, axis=0), _PS, axis=1)


_MASK_HW = _full_mask().astype(np.float32)


def _select_body(m_ref, x_ref, o_ref):
    o_ref[...] = jnp.where(m_ref[...] != 0.0, _MASK_VALUE, x_ref[...])


def kernel(x):
    B, C, H, W = x.shape
    xr = x.reshape(B * C, H, W)
    mask = jnp.asarray(_MASK_HW)
    out = pl.pallas_call(
        _select_body,
        grid=(B * C,),
        in_specs=[
            pl.BlockSpec((H, W), lambda i: (0, 0)),
            pl.BlockSpec((1, H, W), lambda i: (i, 0, 0)),
        ],
        out_specs=pl.BlockSpec((1, H, W), lambda i: (i, 0, 0)),
        out_shape=jax.ShapeDtypeStruct((B * C, H, W), x.dtype),
    )(mask, xr)
    return out.reshape(B, C, H, W)


# speedup vs baseline: 9.7060x; 1.5321x over previous
import numpy as np
import jax
import jax.numpy as jnp
from jax.experimental import pallas as pl
from jax.experimental.pallas import tpu as pltpu

_PS = 16
_H = 512
_W = 512
_HP = _H // _PS
_WP = _W // _PS
_TOTAL = _HP * _WP
_NUM_MASK = int(0.75 * _TOTAL)
_MASK_VALUE = 0.5
_BLOCK_ROWS = 2048


def _full_mask() -> np.ndarray:
    perm = np.asarray(jax.random.permutation(jax.random.key(42), _TOTAL))
    patch_mask = np.zeros(_TOTAL, dtype=bool)
    patch_mask[perm[:_NUM_MASK]] = True
    grid2d = patch_mask.reshape(_HP, _WP)
    return np.repeat(np.repeat(grid2d, _PS, axis=0), _PS, axis=1)


_MASK_BLOCK = np.tile(_full_mask(), (_BLOCK_ROWS // _H, 1)).astype(np.float32)


def _select_body(m_ref, x_ref, o_ref):
    o_ref[...] = jnp.where(m_ref[...] != 0.0, _MASK_VALUE, x_ref[...])


def kernel(x):
    B, C, H, W = x.shape
    rows = B * C * H
    xr = x.reshape(rows, W)
    mask = jnp.asarray(_MASK_BLOCK)
    out = pl.pallas_call(
        _select_body,
        grid=(rows // _BLOCK_ROWS,),
        in_specs=[
            pl.BlockSpec((_BLOCK_ROWS, W), lambda i: (0, 0)),
            pl.BlockSpec((_BLOCK_ROWS, W), lambda i: (i, 0)),
        ],
        out_specs=pl.BlockSpec((_BLOCK_ROWS, W), lambda i: (i, 0)),
        out_shape=jax.ShapeDtypeStruct((rows, W), x.dtype),
        compiler_params=pltpu.CompilerParams(
            dimension_semantics=("parallel",),
        ),
    )(mask, xr)
    return out.reshape(B, C, H, W)


# speedup vs baseline: 9.8128x; 1.0110x over previous
import numpy as np
import jax
import jax.numpy as jnp
from jax.experimental import pallas as pl
from jax.experimental.pallas import tpu as pltpu

_PS = 16
_H = 512
_W = 512
_HP = _H // _PS
_WP = _W // _PS
_TOTAL = _HP * _WP
_NUM_MASK = int(0.75 * _TOTAL)
_MASK_VALUE = 0.5
_BLOCK_ROWS = 4096


def _full_mask() -> np.ndarray:
    perm = np.asarray(jax.random.permutation(jax.random.key(42), _TOTAL))
    patch_mask = np.zeros(_TOTAL, dtype=bool)
    patch_mask[perm[:_NUM_MASK]] = True
    grid2d = patch_mask.reshape(_HP, _WP)
    return np.repeat(np.repeat(grid2d, _PS, axis=0), _PS, axis=1)


_MASK_BLOCK = np.tile(_full_mask(), (_BLOCK_ROWS // _H, 1)).astype(np.float32)


def _select_body(m_ref, x_ref, o_ref):
    o_ref[...] = jnp.where(m_ref[...] != 0.0, _MASK_VALUE, x_ref[...])


def kernel(x):
    B, C, H, W = x.shape
    rows = B * C * H
    xr = x.reshape(rows, W)
    mask = jnp.asarray(_MASK_BLOCK)
    out = pl.pallas_call(
        _select_body,
        grid=(rows // _BLOCK_ROWS,),
        in_specs=[
            pl.BlockSpec((_BLOCK_ROWS, W), lambda i: (0, 0)),
            pl.BlockSpec((_BLOCK_ROWS, W), lambda i: (i, 0)),
        ],
        out_specs=pl.BlockSpec((_BLOCK_ROWS, W), lambda i: (i, 0)),
        out_shape=jax.ShapeDtypeStruct((rows, W), x.dtype),
        compiler_params=pltpu.CompilerParams(
            dimension_semantics=("parallel",),
        ),
    )(mask, xr)
    return out.reshape(B, C, H, W)
